# baseline (speedup 1.0000x reference)
import functools

import jax
import jax.numpy as jnp
from jax import lax
from jax.experimental import pallas as pl
from jax.experimental.pallas import tpu as pltpu
from jax.experimental.pallas import tpu_sc as plsc

_TOKENS = 16384
_HIDDEN = 2048
_EXPERTS = 64
_BT = 2048

_NC, _NS, _L = 2, 16, 16
_NW = _NC * _NS
_TPW = _TOKENS // _NW
_G = _TPW // _L
_NSTREAM = 4
_EPS = _EXPERTS // _NSTREAM


def _logits_body(x_ref, w_ref, b_ref, t_ref, logits_ref):
    x = x_ref[...]
    w = w_ref[...]
    logits = jnp.dot(x, w, preferred_element_type=jnp.float32)
    scale = 1.0 / jnp.abs(t_ref[0, 0])
    logits_ref[...] = (logits + b_ref[...]) * scale


def _tc_logits(hidden_states, gate_w, gate_b, temperature):
    grid = (_TOKENS // _BT,)
    return pl.pallas_call(
        _logits_body,
        grid=grid,
        in_specs=[
            pl.BlockSpec((_BT, _HIDDEN), lambda i: (i, 0)),
            pl.BlockSpec((_HIDDEN, _EXPERTS), lambda i: (0, 0)),
            pl.BlockSpec((1, _EXPERTS), lambda i: (0, 0)),
            pl.BlockSpec((1, 1), lambda i: (0, 0)),
        ],
        out_specs=pl.BlockSpec((_BT, _EXPERTS), lambda i: (i, 0)),
        out_shape=jax.ShapeDtypeStruct((_TOKENS, _EXPERTS), jnp.float32),
    )(
        hidden_states,
        gate_w,
        gate_b.reshape(1, _EXPERTS),
        temperature.reshape(1, 1),
    )


def _merge_top2(a, b):
    a1, ai1, a2, ai2 = a
    b1, bi1, b2, bi2 = b
    a_wins = a1 >= b1
    sa = a2 >= b1
    sb = a1 >= b2
    m1 = jnp.where(a_wins, a1, b1)
    i1 = jnp.where(a_wins, ai1, bi1)
    m2 = jnp.where(a_wins, jnp.where(sa, a2, b1), jnp.where(sb, a1, b2))
    i2 = jnp.where(a_wins, jnp.where(sa, ai2, bi1), jnp.where(sb, ai1, bi2))
    return m1, i1, m2, i2


@functools.partial(
    pl.kernel,
    mesh=plsc.VectorSubcoreMesh(core_axis_name="c", subcore_axis_name="s"),
    out_type=[
        jax.ShapeDtypeStruct((_TOKENS * 2,), jnp.float32),
        jax.ShapeDtypeStruct((_TOKENS * 2,), jnp.int32),
    ],
    scratch_types=[
        pltpu.VMEM((_TPW * _EXPERTS,), jnp.float32),
        pltpu.VMEM((_TPW * 2,), jnp.float32),
        pltpu.VMEM((_TPW * 2,), jnp.int32),
    ],
    compiler_params=pltpu.CompilerParams(needs_layout_passes=False),
)
def _sc_route(logits_hbm, wout_hbm, eout_hbm, lg_v, w_v, e_v):
    wid = lax.axis_index("s") * _NC + lax.axis_index("c")
    base = wid * _TPW
    pltpu.sync_copy(logits_hbm.at[pl.ds(base * _EXPERTS, _TPW * _EXPERTS)], lg_v)

    lane = lax.iota(jnp.int32, _L)
    neg = jnp.full((_L,), -3.0e38, jnp.float32)

    def group(g, _):
        rows = g * _L + lane
        flat0 = rows * _EXPERTS
        pair0 = rows * 2
        zero16 = jnp.zeros((_L,), jnp.int32)
        streams = []
        for j in range(_NSTREAM):
            m1, i1, m2, i2 = neg, zero16, neg, zero16
            for i in range(_EPS):
                e = j * _EPS + i
                e16 = jnp.full((_L,), e, jnp.int32)
                v = plsc.load_gather(lg_v, [flat0 + e])
                gt1 = v > m1
                gt2 = jnp.logical_and(v > m2, jnp.logical_not(gt1))
                i2 = jnp.where(gt1, i1, jnp.where(gt2, e16, i2))
                m2 = jnp.where(gt1, m1, jnp.where(gt2, v, m2))
                i1 = jnp.where(gt1, e16, i1)
                m1 = jnp.where(gt1, v, m1)
            streams.append((m1, i1, m2, i2))
        m1, i1, m2, i2 = _merge_top2(
            _merge_top2(streams[0], streams[1]),
            _merge_top2(streams[2], streams[3]),
        )
        ex = jnp.exp(m2 - m1)
        den = 1.0 + ex
        plsc.store_scatter(w_v, [pair0], 1.0 / den)
        plsc.store_scatter(w_v, [pair0 + 1], ex / den)
        plsc.store_scatter(e_v, [pair0], i1)
        plsc.store_scatter(e_v, [pair0 + 1], i2)
        return 0

    lax.fori_loop(0, _G, group, 0)
    pltpu.sync_copy(w_v, wout_hbm.at[pl.ds(base * 2, _TPW * 2)])
    pltpu.sync_copy(e_v, eout_hbm.at[pl.ds(base * 2, _TPW * 2)])


def kernel(hidden_states, gate_w, gate_b, temperature, noise_w, noise_b):
    del noise_w, noise_b
    router_logits = _tc_logits(hidden_states, gate_w, gate_b, temperature)
    wflat, eflat = _sc_route(router_logits.reshape(-1))
    router_weights = wflat.reshape(_TOKENS, 2)
    select_experts = eflat.reshape(_TOKENS, 2)
    return (router_logits, router_weights, select_experts)


# speedup vs baseline: 1.0559x; 1.0559x over previous
import functools

import jax
import jax.numpy as jnp
from jax import lax
from jax.experimental import pallas as pl
from jax.experimental.pallas import tpu as pltpu
from jax.experimental.pallas import tpu_sc as plsc

_TOKENS = 16384
_HIDDEN = 2048
_EXPERTS = 64
_PADE = 128
_BT = 2048

_NC, _NS, _L = 2, 16, 16
_NW = _NC * _NS
_TPW = _TOKENS // _NW
_G = _TPW // _L
_NSTREAM = 4
_EPS = _EXPERTS // _NSTREAM
_STRIDE = 65


def _logits_body(x_ref, w_ref, b_ref, t_ref, logits_ref, aux_ref):
    x = x_ref[...]
    w = w_ref[...]
    logits = jnp.dot(x, w, preferred_element_type=jnp.float32)
    scale = 1.0 / jnp.abs(t_ref[0, 0])
    logits = (logits + b_ref[...]) * scale
    logits_ref[...] = logits
    aux_ref[:, 0:_EXPERTS] = logits


def _tc_logits(hidden_states, gate_w, gate_b, temperature):
    grid = (_TOKENS // _BT,)
    return pl.pallas_call(
        _logits_body,
        grid=grid,
        in_specs=[
            pl.BlockSpec((_BT, _HIDDEN), lambda i: (i, 0)),
            pl.BlockSpec((_HIDDEN, _EXPERTS), lambda i: (0, 0)),
            pl.BlockSpec((1, _EXPERTS), lambda i: (0, 0)),
            pl.BlockSpec((1, 1), lambda i: (0, 0)),
        ],
        out_specs=[
            pl.BlockSpec((_BT, _EXPERTS), lambda i: (i, 0)),
            pl.BlockSpec((_BT, _PADE), lambda i: (i, 0)),
        ],
        out_shape=[
            jax.ShapeDtypeStruct((_TOKENS, _EXPERTS), jnp.float32),
            jax.ShapeDtypeStruct((_TOKENS, _PADE), jnp.float32),
        ],
    )(
        hidden_states,
        gate_w,
        gate_b.reshape(1, _EXPERTS),
        temperature.reshape(1, 1),
    )


def _merge_top2(a, b):
    a1, ai1, a2, ai2 = a
    b1, bi1, b2, bi2 = b
    a_wins = a1 >= b1
    sa = a2 >= b1
    sb = a1 >= b2
    m1 = jnp.where(a_wins, a1, b1)
    i1 = jnp.where(a_wins, ai1, bi1)
    m2 = jnp.where(a_wins, jnp.where(sa, a2, b1), jnp.where(sb, a1, b2))
    i2 = jnp.where(a_wins, jnp.where(sa, ai2, bi1), jnp.where(sb, ai1, bi2))
    return m1, i1, m2, i2


@functools.partial(
    pl.kernel,
    mesh=plsc.VectorSubcoreMesh(core_axis_name="c", subcore_axis_name="s"),
    out_type=[
        jax.ShapeDtypeStruct((_TOKENS * 2,), jnp.float32),
        jax.ShapeDtypeStruct((_TOKENS * 2,), jnp.int32),
    ],
    scratch_types=[
        pltpu.VMEM((_TPW * _PADE,), jnp.float32),
        pltpu.VMEM((_TPW * _STRIDE,), jnp.float32),
        pltpu.VMEM((_TPW * 2,), jnp.float32),
        pltpu.VMEM((_TPW * 2,), jnp.int32),
    ],
    compiler_params=pltpu.CompilerParams(needs_layout_passes=False),
)
def _sc_route(aux_hbm, wout_hbm, eout_hbm, slab_v, str_v, w_v, e_v):
    wid = lax.axis_index("s") * _NC + lax.axis_index("c")
    base = wid * _TPW
    pltpu.sync_copy(aux_hbm.at[pl.ds(base * _PADE, _TPW * _PADE)], slab_v)

    def restride(t, _):
        for j in range(0, _EXPERTS, _L):
            str_v[pl.ds(t * _STRIDE + j, _L)] = slab_v[pl.ds(t * _PADE + j, _L)]
        return 0

    lax.fori_loop(0, _TPW, restride, 0, unroll=8)

    lane = lax.iota(jnp.int32, _L)
    neg = jnp.full((_L,), -3.0e38, jnp.float32)

    def group(g, _):
        rows = g * _L + lane
        flat0 = rows * _STRIDE
        pair0 = rows * 2
        zero16 = jnp.zeros((_L,), jnp.int32)
        streams = []
        for j in range(_NSTREAM):
            m1, i1, m2, i2 = neg, zero16, neg, zero16
            for i in range(_EPS):
                e = j * _EPS + i
                e16 = jnp.full((_L,), e, jnp.int32)
                v = plsc.load_gather(str_v, [flat0 + e])
                gt1 = v > m1
                gt2 = jnp.logical_and(v > m2, jnp.logical_not(gt1))
                i2 = jnp.where(gt1, i1, jnp.where(gt2, e16, i2))
                m2 = jnp.where(gt1, m1, jnp.where(gt2, v, m2))
                i1 = jnp.where(gt1, e16, i1)
                m1 = jnp.where(gt1, v, m1)
            streams.append((m1, i1, m2, i2))
        m1, i1, m2, i2 = _merge_top2(
            _merge_top2(streams[0], streams[1]),
            _merge_top2(streams[2], streams[3]),
        )
        ex = jnp.exp(m2 - m1)
        den = 1.0 + ex
        plsc.store_scatter(w_v, [pair0], 1.0 / den)
        plsc.store_scatter(w_v, [pair0 + 1], ex / den)
        plsc.store_scatter(e_v, [pair0], i1)
        plsc.store_scatter(e_v, [pair0 + 1], i2)
        return 0

    lax.fori_loop(0, _G, group, 0)
    pltpu.sync_copy(w_v, wout_hbm.at[pl.ds(base * 2, _TPW * 2)])
    pltpu.sync_copy(e_v, eout_hbm.at[pl.ds(base * 2, _TPW * 2)])


def kernel(hidden_states, gate_w, gate_b, temperature, noise_w, noise_b):
    del noise_w, noise_b
    router_logits, aux = _tc_logits(hidden_states, gate_w, gate_b, temperature)
    wflat, eflat = _sc_route(aux.reshape(-1))
    router_weights = wflat.reshape(_TOKENS, 2)
    select_experts = eflat.reshape(_TOKENS, 2)
    return (router_logits, router_weights, select_experts)


# speedup vs baseline: 1.0989x; 1.0408x over previous
import functools

import jax
import jax.numpy as jnp
from jax import lax
from jax.experimental import pallas as pl
from jax.experimental.pallas import tpu as pltpu
from jax.experimental.pallas import tpu_sc as plsc

_TOKENS = 16384
_HIDDEN = 2048
_EXPERTS = 64
_PADE = 128
_BT = 2048

_NC, _NS, _L = 2, 16, 16
_NW = _NC * _NS
_TPW = _TOKENS // _NW
_G = _TPW // _L
_NSTREAM = 4
_EPS = _EXPERTS // _NSTREAM
_STRIDE = 65


def _logits_body(x_ref, w_ref, b_ref, t_ref, logits_ref, aux_ref):
    x = x_ref[...]
    w = w_ref[...]
    logits = jnp.dot(x, w, preferred_element_type=jnp.float32)
    scale = 1.0 / jnp.abs(t_ref[0, 0])
    logits = (logits + b_ref[...]) * scale
    logits_ref[...] = logits
    aux_ref[...] = jnp.swapaxes(logits, 0, 1)


def _tc_logits(hidden_states, gate_w, gate_b, temperature):
    grid = (_TOKENS // _BT,)
    return pl.pallas_call(
        _logits_body,
        grid=grid,
        in_specs=[
            pl.BlockSpec((_BT, _HIDDEN), lambda i: (i, 0)),
            pl.BlockSpec((_HIDDEN, _EXPERTS), lambda i: (0, 0)),
            pl.BlockSpec((1, _EXPERTS), lambda i: (0, 0)),
            pl.BlockSpec((1, 1), lambda i: (0, 0)),
        ],
        out_specs=[
            pl.BlockSpec((_BT, _EXPERTS), lambda i: (i, 0)),
            pl.BlockSpec((_EXPERTS, _BT), lambda i: (0, i)),
        ],
        out_shape=[
            jax.ShapeDtypeStruct((_TOKENS, _EXPERTS), jnp.float32),
            jax.ShapeDtypeStruct((_EXPERTS, _TOKENS), jnp.float32),
        ],
    )(
        hidden_states,
        gate_w,
        gate_b.reshape(1, _EXPERTS),
        temperature.reshape(1, 1),
    )


def _merge_top2(a, b):
    a1, ai1, a2, ai2 = a
    b1, bi1, b2, bi2 = b
    a_wins = a1 >= b1
    sa = a2 >= b1
    sb = a1 >= b2
    m1 = jnp.where(a_wins, a1, b1)
    i1 = jnp.where(a_wins, ai1, bi1)
    m2 = jnp.where(a_wins, jnp.where(sa, a2, b1), jnp.where(sb, a1, b2))
    i2 = jnp.where(a_wins, jnp.where(sa, ai2, bi1), jnp.where(sb, ai1, bi2))
    return m1, i1, m2, i2


@functools.partial(
    pl.kernel,
    mesh=plsc.VectorSubcoreMesh(core_axis_name="c", subcore_axis_name="s"),
    out_type=[
        jax.ShapeDtypeStruct((_TOKENS * 2,), jnp.float32),
        jax.ShapeDtypeStruct((_TOKENS * 2,), jnp.int32),
    ],
    scratch_types=[
        pltpu.VMEM((_TPW * _EXPERTS,), jnp.float32),
        pltpu.VMEM((_TPW * 2,), jnp.float32),
        pltpu.VMEM((_TPW * 2,), jnp.int32),
        pltpu.SemaphoreType.DMA,
    ],
    compiler_params=pltpu.CompilerParams(needs_layout_passes=False),
)
def _sc_route(aux_hbm, wout_hbm, eout_hbm, slab_v, w_v, e_v, sem):
    wid = lax.axis_index("s") * _NC + lax.axis_index("c")
    base = wid * _TPW
    copies = [
        pltpu.async_copy(
            aux_hbm.at[pl.ds(e * _TOKENS + base, _TPW)],
            slab_v.at[pl.ds(e * _TPW, _TPW)],
            sem,
        )
        for e in range(_EXPERTS)
    ]
    for cp in copies:
        cp.wait()

    lane = lax.iota(jnp.int32, _L)
    neg = jnp.full((_L,), -3.0e38, jnp.float32)

    def group(g, _):
        rows = g * _L + lane
        t0 = g * _L
        pair0 = rows * 2
        zero16 = jnp.zeros((_L,), jnp.int32)
        streams = []
        for j in range(_NSTREAM):
            m1, i1, m2, i2 = neg, zero16, neg, zero16
            for i in range(_EPS):
                e = j * _EPS + i
                e16 = jnp.full((_L,), e, jnp.int32)
                v = slab_v[pl.ds(e * _TPW + t0, _L)]
                gt1 = v > m1
                gt2 = jnp.logical_and(v > m2, jnp.logical_not(gt1))
                i2 = jnp.where(gt1, i1, jnp.where(gt2, e16, i2))
                m2 = jnp.where(gt1, m1, jnp.where(gt2, v, m2))
                i1 = jnp.where(gt1, e16, i1)
                m1 = jnp.where(gt1, v, m1)
            streams.append((m1, i1, m2, i2))
        m1, i1, m2, i2 = _merge_top2(
            _merge_top2(streams[0], streams[1]),
            _merge_top2(streams[2], streams[3]),
        )
        ex = jnp.exp(m2 - m1)
        den = 1.0 + ex
        plsc.store_scatter(w_v, [pair0], 1.0 / den)
        plsc.store_scatter(w_v, [pair0 + 1], ex / den)
        plsc.store_scatter(e_v, [pair0], i1)
        plsc.store_scatter(e_v, [pair0 + 1], i2)
        return 0

    lax.fori_loop(0, _G, group, 0)
    pltpu.sync_copy(w_v, wout_hbm.at[pl.ds(base * 2, _TPW * 2)])
    pltpu.sync_copy(e_v, eout_hbm.at[pl.ds(base * 2, _TPW * 2)])


def kernel(hidden_states, gate_w, gate_b, temperature, noise_w, noise_b):
    del noise_w, noise_b
    router_logits, aux = _tc_logits(hidden_states, gate_w, gate_b, temperature)
    wflat, eflat = _sc_route(aux.reshape(-1))
    router_weights = wflat.reshape(_TOKENS, 2)
    select_experts = eflat.reshape(_TOKENS, 2)
    return (router_logits, router_weights, select_experts)


# speedup vs baseline: 1.1506x; 1.0470x over previous
import functools

import jax
import jax.numpy as jnp
from jax import lax
from jax.experimental import pallas as pl
from jax.experimental.pallas import tpu as pltpu
from jax.experimental.pallas import tpu_sc as plsc

_TOKENS = 16384
_HIDDEN = 2048
_EXPERTS = 64
_PADE = 128
_BT = 2048

_NC, _NS, _L = 2, 16, 16
_NW = _NC * _NS
_TPW = _TOKENS // _NW
_G = _TPW // _L
_NSTREAM = 4
_EPS = _EXPERTS // _NSTREAM
_STRIDE = 65


def _logits_body(x_ref, w_ref, b_ref, t_ref, logits_ref, aux_ref):
    x = x_ref[...]
    w = w_ref[...]
    logits = jnp.dot(x, w, preferred_element_type=jnp.float32)
    scale = 1.0 / jnp.abs(t_ref[0, 0])
    logits = (logits + b_ref[...]) * scale
    logits_ref[...] = logits
    aux_ref[...] = jnp.swapaxes(logits, 0, 1)


def _tc_logits(hidden_states, gate_w, gate_b, temperature):
    grid = (_TOKENS // _BT,)
    return pl.pallas_call(
        _logits_body,
        grid=grid,
        in_specs=[
            pl.BlockSpec((_BT, _HIDDEN), lambda i: (i, 0)),
            pl.BlockSpec((_HIDDEN, _EXPERTS), lambda i: (0, 0)),
            pl.BlockSpec((1, _EXPERTS), lambda i: (0, 0)),
            pl.BlockSpec((1, 1), lambda i: (0, 0)),
        ],
        out_specs=[
            pl.BlockSpec((_BT, _EXPERTS), lambda i: (i, 0)),
            pl.BlockSpec((_EXPERTS, _BT), lambda i: (0, i)),
        ],
        out_shape=[
            jax.ShapeDtypeStruct((_TOKENS, _EXPERTS), jnp.float32),
            jax.ShapeDtypeStruct((_EXPERTS, _TOKENS), jnp.float32),
        ],
    )(
        hidden_states,
        gate_w,
        gate_b.reshape(1, _EXPERTS),
        temperature.reshape(1, 1),
    )


def _merge_top2(a, b):
    a1, ai1, a2, ai2 = a
    b1, bi1, b2, bi2 = b
    a_wins = a1 >= b1
    sa = a2 >= b1
    sb = a1 >= b2
    m1 = jnp.where(a_wins, a1, b1)
    i1 = jnp.where(a_wins, ai1, bi1)
    m2 = jnp.where(a_wins, jnp.where(sa, a2, b1), jnp.where(sb, a1, b2))
    i2 = jnp.where(a_wins, jnp.where(sa, ai2, bi1), jnp.where(sb, ai1, bi2))
    return m1, i1, m2, i2


@functools.partial(
    pl.kernel,
    mesh=plsc.VectorSubcoreMesh(core_axis_name="c", subcore_axis_name="s"),
    out_type=[
        jax.ShapeDtypeStruct((_TOKENS * 2,), jnp.float32),
        jax.ShapeDtypeStruct((_TOKENS * 2,), jnp.int32),
    ],
    scratch_types=[
        pltpu.VMEM((_TPW * _EXPERTS,), jnp.float32),
        pltpu.VMEM((_TPW * 2,), jnp.float32),
        pltpu.VMEM((_TPW * 2,), jnp.int32),
        pltpu.SemaphoreType.DMA,
    ],
    compiler_params=pltpu.CompilerParams(needs_layout_passes=False),
)
def _sc_route(aux_hbm, wout_hbm, eout_hbm, slab_v, w_v, e_v, sem):
    wid = lax.axis_index("s") * _NC + lax.axis_index("c")
    base = wid * _TPW
    copies = [
        pltpu.async_copy(
            aux_hbm.at[e, pl.ds(base, _TPW)],
            slab_v.at[pl.ds(e * _TPW, _TPW)],
            sem,
        )
        for e in range(_EXPERTS)
    ]
    for cp in copies:
        cp.wait()

    lane = lax.iota(jnp.int32, _L)
    neg = jnp.full((_L,), -3.0e38, jnp.float32)

    def group(g, _):
        rows = g * _L + lane
        t0 = g * _L
        pair0 = rows * 2
        zero16 = jnp.zeros((_L,), jnp.int32)
        streams = []
        for j in range(_NSTREAM):
            m1, i1, m2, i2 = neg, zero16, neg, zero16
            for i in range(_EPS):
                e = j * _EPS + i
                e16 = jnp.full((_L,), e, jnp.int32)
                v = slab_v[pl.ds(e * _TPW + t0, _L)]
                gt1 = v > m1
                gt2 = jnp.logical_and(v > m2, jnp.logical_not(gt1))
                i2 = jnp.where(gt1, i1, jnp.where(gt2, e16, i2))
                m2 = jnp.where(gt1, m1, jnp.where(gt2, v, m2))
                i1 = jnp.where(gt1, e16, i1)
                m1 = jnp.where(gt1, v, m1)
            streams.append((m1, i1, m2, i2))
        m1, i1, m2, i2 = _merge_top2(
            _merge_top2(streams[0], streams[1]),
            _merge_top2(streams[2], streams[3]),
        )
        ex = jnp.exp(m2 - m1)
        den = 1.0 + ex
        plsc.store_scatter(w_v, [pair0], 1.0 / den)
        plsc.store_scatter(w_v, [pair0 + 1], ex / den)
        plsc.store_scatter(e_v, [pair0], i1)
        plsc.store_scatter(e_v, [pair0 + 1], i2)
        return 0

    lax.fori_loop(0, _G, group, 0)
    pltpu.sync_copy(w_v, wout_hbm.at[pl.ds(base * 2, _TPW * 2)])
    pltpu.sync_copy(e_v, eout_hbm.at[pl.ds(base * 2, _TPW * 2)])


def kernel(hidden_states, gate_w, gate_b, temperature, noise_w, noise_b):
    del noise_w, noise_b
    router_logits, aux = _tc_logits(hidden_states, gate_w, gate_b, temperature)
    wflat, eflat = _sc_route(aux)
    router_weights = wflat.reshape(_TOKENS, 2)
    select_experts = eflat.reshape(_TOKENS, 2)
    return (router_logits, router_weights, select_experts)


# speedup vs baseline: 1.5643x; 1.3596x over previous
import functools

import jax
import jax.numpy as jnp
from jax import lax
from jax.experimental import pallas as pl
from jax.experimental.pallas import tpu as pltpu
from jax.experimental.pallas import tpu_sc as plsc

_TOKENS = 16384
_HIDDEN = 2048
_EXPERTS = 64
_PADE = 128
_BT = 2048

_NC, _NS, _L = 2, 16, 16
_NW = _NC * _NS
_TPW = _TOKENS // _NW
_G = _TPW // _L
_NSTREAM = 4
_EPS = _EXPERTS // _NSTREAM
_STRIDE = 65


def _logits_body(x_ref, w_ref, b_ref, t_ref, logits_ref, aux_ref):
    x = x_ref[...]
    w = w_ref[...]
    logits = jnp.dot(x, w, preferred_element_type=jnp.float32)
    scale = 1.0 / jnp.abs(t_ref[0, 0])
    logits = (logits + b_ref[...]) * scale
    logits_ref[...] = logits
    aux_ref[...] = jnp.swapaxes(logits, 0, 1)


def _tc_logits(hidden_states, gate_w, gate_b, temperature):
    grid = (_TOKENS // _BT,)
    return pl.pallas_call(
        _logits_body,
        grid=grid,
        in_specs=[
            pl.BlockSpec((_BT, _HIDDEN), lambda i: (i, 0)),
            pl.BlockSpec((_HIDDEN, _EXPERTS), lambda i: (0, 0)),
            pl.BlockSpec((1, _EXPERTS), lambda i: (0, 0)),
            pl.BlockSpec((1, 1), lambda i: (0, 0)),
        ],
        out_specs=[
            pl.BlockSpec((_BT, _EXPERTS), lambda i: (i, 0)),
            pl.BlockSpec((_EXPERTS, _BT), lambda i: (0, i)),
        ],
        out_shape=[
            jax.ShapeDtypeStruct((_TOKENS, _EXPERTS), jnp.float32),
            jax.ShapeDtypeStruct((_EXPERTS, _TOKENS), jnp.float32),
        ],
    )(
        hidden_states,
        gate_w,
        gate_b.reshape(1, _EXPERTS),
        temperature.reshape(1, 1),
    )


def _merge_top2(a, b):
    a1, ai1, a2, ai2 = a
    b1, bi1, b2, bi2 = b
    a_wins = a1 >= b1
    sa = a2 >= b1
    sb = a1 >= b2
    m1 = jnp.where(a_wins, a1, b1)
    i1 = jnp.where(a_wins, ai1, bi1)
    m2 = jnp.where(a_wins, jnp.where(sa, a2, b1), jnp.where(sb, a1, b2))
    i2 = jnp.where(a_wins, jnp.where(sa, ai2, bi1), jnp.where(sb, ai1, bi2))
    return m1, i1, m2, i2


@functools.partial(
    pl.kernel,
    mesh=plsc.VectorSubcoreMesh(core_axis_name="c", subcore_axis_name="s"),
    out_type=[
        jax.ShapeDtypeStruct((_TOKENS,), jnp.float32),
        jax.ShapeDtypeStruct((_TOKENS,), jnp.float32),
        jax.ShapeDtypeStruct((_TOKENS,), jnp.int32),
        jax.ShapeDtypeStruct((_TOKENS,), jnp.int32),
    ],
    scratch_types=[
        pltpu.VMEM((_TPW * _EXPERTS,), jnp.float32),
        pltpu.VMEM((_TPW,), jnp.float32),
        pltpu.VMEM((_TPW,), jnp.float32),
        pltpu.VMEM((_TPW,), jnp.int32),
        pltpu.VMEM((_TPW,), jnp.int32),
        pltpu.SemaphoreType.DMA,
    ],
    compiler_params=pltpu.CompilerParams(needs_layout_passes=False),
)
def _sc_route(aux_hbm, w1_hbm, w2_hbm, e1_hbm, e2_hbm, slab_v, w1_v, w2_v, e1_v, e2_v, sem):
    wid = lax.axis_index("s") * _NC + lax.axis_index("c")
    base = wid * _TPW
    copies = [
        pltpu.async_copy(
            aux_hbm.at[e, pl.ds(base, _TPW)],
            slab_v.at[pl.ds(e * _TPW, _TPW)],
            sem,
        )
        for e in range(_EXPERTS)
    ]
    for cp in copies:
        cp.wait()

    lane = lax.iota(jnp.int32, _L)
    neg = jnp.full((_L,), -3.0e38, jnp.float32)

    def group(g, _):
        t0 = g * _L
        zero16 = jnp.zeros((_L,), jnp.int32)
        streams = []
        for j in range(_NSTREAM):
            m1, i1, m2, i2 = neg, zero16, neg, zero16
            for i in range(_EPS):
                e = j * _EPS + i
                e16 = jnp.full((_L,), e, jnp.int32)
                v = slab_v[pl.ds(e * _TPW + t0, _L)]
                gt1 = v > m1
                gt2 = jnp.logical_and(v > m2, jnp.logical_not(gt1))
                i2 = jnp.where(gt1, i1, jnp.where(gt2, e16, i2))
                m2 = jnp.where(gt1, m1, jnp.where(gt2, v, m2))
                i1 = jnp.where(gt1, e16, i1)
                m1 = jnp.where(gt1, v, m1)
            streams.append((m1, i1, m2, i2))
        m1, i1, m2, i2 = _merge_top2(
            _merge_top2(streams[0], streams[1]),
            _merge_top2(streams[2], streams[3]),
        )
        ex = jnp.exp(m2 - m1)
        den = 1.0 + ex
        w1_v[pl.ds(t0, _L)] = 1.0 / den
        w2_v[pl.ds(t0, _L)] = ex / den
        e1_v[pl.ds(t0, _L)] = i1
        e2_v[pl.ds(t0, _L)] = i2
        return 0

    lax.fori_loop(0, _G, group, 0)
    pltpu.sync_copy(w1_v, w1_hbm.at[pl.ds(base, _TPW)])
    pltpu.sync_copy(w2_v, w2_hbm.at[pl.ds(base, _TPW)])
    pltpu.sync_copy(e1_v, e1_hbm.at[pl.ds(base, _TPW)])
    pltpu.sync_copy(e2_v, e2_hbm.at[pl.ds(base, _TPW)])


def kernel(hidden_states, gate_w, gate_b, temperature, noise_w, noise_b):
    del noise_w, noise_b
    router_logits, aux = _tc_logits(hidden_states, gate_w, gate_b, temperature)
    w1, w2, e1, e2 = _sc_route(aux)
    router_weights = jnp.stack([w1, w2], axis=-1)
    select_experts = jnp.stack([e1, e2], axis=-1)
    return (router_logits, router_weights, select_experts)
